# baseline (speedup 1.0000x reference)
import functools
import math

import jax
import jax.numpy as jnp
from jax import lax
from jax.experimental import pallas as pl
from jax.experimental.pallas import tpu as pltpu

_INV_SQRT2 = 1.0 / math.sqrt(2.0)


def _gelu_exact(x):
    return 0.5 * x * (1.0 + lax.erf(x * _INV_SQRT2))


def _ffn_kernel(x_ref, w1_ref, b1_ref, w2_ref, b2_ref, o_ref, *, th):
    xb = x_ref[...].astype(jnp.bfloat16)
    n_h = w1_ref.shape[1] // th
    acc = jnp.broadcast_to(b2_ref[...].astype(jnp.float32), o_ref.shape)
    for c in range(n_h):
        w1c = w1_ref[:, c * th:(c + 1) * th].astype(jnp.bfloat16)
        h = jnp.dot(xb, w1c, preferred_element_type=jnp.float32)
        h = _gelu_exact(h + b1_ref[:, c * th:(c + 1) * th].astype(jnp.float32))
        w2c = w2_ref[c * th:(c + 1) * th, :].astype(jnp.bfloat16)
        acc = jnp.dot(h.astype(jnp.bfloat16), w2c,
                      preferred_element_type=jnp.float32) + acc
    o_ref[...] = acc.astype(o_ref.dtype)


def kernel(x, w1, b1, w2, b2):
    batch, seq, dim = x.shape
    hidden = w1.shape[1]
    M = batch * seq
    x2d = x.reshape(M, dim)

    b1r = b1.reshape(1, hidden).astype(jnp.float32)
    b2r = b2.reshape(1, dim).astype(jnp.float32)

    TM = 1024
    Mp = -(-M // (2 * TM)) * (2 * TM)
    if Mp != M:
        x2d = jnp.pad(x2d, ((0, Mp - M), (0, 0)))

    th = 512 if hidden % 512 == 0 else hidden
    cost = pl.CostEstimate(
        flops=int(4 * Mp * dim * hidden),
        transcendentals=int(Mp * hidden),
        bytes_accessed=int(4 * Mp * dim * 2 + 2 * (dim * hidden * 4)),
    )

    out2d = pl.pallas_call(
        functools.partial(_ffn_kernel, th=th),
        out_shape=jax.ShapeDtypeStruct((Mp, dim), x.dtype),
        grid=(Mp // TM,),
        in_specs=[
            pl.BlockSpec((TM, dim), lambda i: (i, 0)),
            pl.BlockSpec((dim, hidden), lambda i: (0, 0)),
            pl.BlockSpec((1, hidden), lambda i: (0, 0)),
            pl.BlockSpec((hidden, dim), lambda i: (0, 0)),
            pl.BlockSpec((1, dim), lambda i: (0, 0)),
        ],
        out_specs=pl.BlockSpec((TM, dim), lambda i: (i, 0)),
        compiler_params=pltpu.CompilerParams(
            dimension_semantics=("parallel",),
            vmem_limit_bytes=61 * 1024 * 1024,
        ),
        cost_estimate=cost,
    )(x2d, w1, b1r, w2, b2r)

    if Mp != M:
        out2d = out2d[:M]
    return out2d.reshape(batch, seq, dim)


# speedup vs baseline: 1.0787x; 1.0787x over previous
import functools
import math

import jax
import jax.numpy as jnp
from jax import lax
from jax.experimental import pallas as pl
from jax.experimental.pallas import tpu as pltpu

_INV_SQRT2 = 1.0 / math.sqrt(2.0)


def _gelu_exact(x):
    return 0.5 * x * (1.0 + lax.erf(x * _INV_SQRT2))


def _ffn_kernel(x_ref, w1_ref, b1_ref, w2_ref, b2_ref, o_ref, *, th):
    xb = x_ref[...].astype(jnp.bfloat16)
    n_h = w1_ref.shape[1] // th
    acc = jnp.broadcast_to(b2_ref[...].astype(jnp.float32), o_ref.shape)
    for c in range(n_h):
        w1c = w1_ref[:, c * th:(c + 1) * th].astype(jnp.bfloat16)
        h = jnp.dot(xb, w1c, preferred_element_type=jnp.float32)
        h = _gelu_exact(h + b1_ref[:, c * th:(c + 1) * th].astype(jnp.float32))
        w2c = w2_ref[c * th:(c + 1) * th, :].astype(jnp.bfloat16)
        acc = jnp.dot(h.astype(jnp.bfloat16), w2c,
                      preferred_element_type=jnp.float32) + acc
    o_ref[...] = acc.astype(o_ref.dtype)


def kernel(x, w1, b1, w2, b2):
    batch, seq, dim = x.shape
    hidden = w1.shape[1]
    M = batch * seq
    x2d = x.reshape(M, dim)

    b1r = b1.reshape(1, hidden).astype(jnp.float32)
    b2r = b2.reshape(1, dim).astype(jnp.float32)

    TM = 1024
    Mp = -(-M // (2 * TM)) * (2 * TM)
    if Mp != M:
        x2d = jnp.pad(x2d, ((0, Mp - M), (0, 0)))

    th = 2048 if hidden % 2048 == 0 else hidden
    cost = pl.CostEstimate(
        flops=int(4 * Mp * dim * hidden),
        transcendentals=int(Mp * hidden),
        bytes_accessed=int(4 * Mp * dim * 2 + 2 * (dim * hidden * 4)),
    )

    out2d = pl.pallas_call(
        functools.partial(_ffn_kernel, th=th),
        out_shape=jax.ShapeDtypeStruct((Mp, dim), x.dtype),
        grid=(Mp // TM,),
        in_specs=[
            pl.BlockSpec((TM, dim), lambda i: (i, 0)),
            pl.BlockSpec((dim, hidden), lambda i: (0, 0)),
            pl.BlockSpec((1, hidden), lambda i: (0, 0)),
            pl.BlockSpec((hidden, dim), lambda i: (0, 0)),
            pl.BlockSpec((1, dim), lambda i: (0, 0)),
        ],
        out_specs=pl.BlockSpec((TM, dim), lambda i: (i, 0)),
        compiler_params=pltpu.CompilerParams(
            dimension_semantics=("parallel",),
            vmem_limit_bytes=61 * 1024 * 1024,
        ),
        cost_estimate=cost,
    )(x2d, w1, b1r, w2, b2r)

    if Mp != M:
        out2d = out2d[:M]
    return out2d.reshape(batch, seq, dim)
